# baseline (speedup 1.0000x reference)
import functools

import jax
import jax.numpy as jnp
from jax import lax
from jax.experimental import pallas as pl
from jax.experimental.pallas import tpu as pltpu
from jax.experimental.pallas import tpu_sc as plsc

N = 10000
E = 320000
D = 128
K = 32
L = 3

NC = 2
NS = 16
NW = NC * NS
CHUNK = 128
CPW = 80
HALF = CPW // 2
NCHUNK = NW * CPW
EPAD = NCHUNK * CHUNK
NP = 10240
ROWS_PER_TILE = NP // NS
DEGW = 128


@functools.lru_cache(maxsize=1)
def _sc_kernels():
    mesh = plsc.VectorSubcoreMesh(
        core_axis_name="c", subcore_axis_name="s", num_cores=NC)

    @functools.partial(
        pl.kernel,
        mesh=mesh,
        out_type=jax.ShapeDtypeStruct((NC, NP, DEGW), jnp.float32),
        scratch_types=[
            pltpu.VMEM((CPW, CHUNK), jnp.int32),
            pltpu.VMEM((CHUNK, DEGW), jnp.float32),
            pltpu.VMEM_SHARED((NP, DEGW), jnp.float32),
        ],
    )
    def sc_degree(dst_hbm, ones_hbm, zeros_hbm, out_hbm, dstv, onesv, acc):
        cid = lax.axis_index("c")
        sid = lax.axis_index("s")
        wid = sid * NC + cid
        base_row = sid * ROWS_PER_TILE
        pltpu.sync_copy(zeros_hbm, acc.at[pl.ds(base_row, ROWS_PER_TILE)])
        pltpu.sync_copy(ones_hbm, onesv)
        pltpu.sync_copy(dst_hbm.at[pl.ds(wid * CPW, CPW)], dstv)
        plsc.subcore_barrier()

        def body(i, carry):
            pltpu.sync_copy(onesv, acc.at[dstv.at[i]], add=True)
            return carry

        lax.fori_loop(0, CPW, body, 0)
        plsc.subcore_barrier()
        pltpu.sync_copy(
            acc.at[pl.ds(base_row, ROWS_PER_TILE)],
            out_hbm.at[cid, pl.ds(base_row, ROWS_PER_TILE)],
        )

    @functools.partial(
        pl.kernel,
        mesh=mesh,
        out_type=jax.ShapeDtypeStruct((NC, NP, D), jnp.float32),
        scratch_types=[
            pltpu.VMEM((HALF, CHUNK), jnp.int32),
            pltpu.VMEM((HALF, CHUNK), jnp.int32),
            pltpu.VMEM((CHUNK, D), jnp.float32),
            pltpu.VMEM((CHUNK, D), jnp.float32),
            pltpu.VMEM_SHARED((NP, D), jnp.float32),
            pltpu.SemaphoreType.DMA,
            pltpu.SemaphoreType.DMA,
        ],
    )
    def sc_aggregate(hk_hbm, src_hbm, dst_hbm, zeros_hbm, out_hbm,
                     srcv, dstv, rows0, rows1, acc, sem0, sem1):
        cid = lax.axis_index("c")
        sid = lax.axis_index("s")
        wid = sid * NC + cid
        base_row = sid * ROWS_PER_TILE
        pltpu.sync_copy(zeros_hbm, acc.at[pl.ds(base_row, ROWS_PER_TILE)])
        plsc.subcore_barrier()

        bufs = ((rows0, sem0), (rows1, sem1))
        for half in range(2):
            chunk0 = wid * CPW + half * HALF
            pltpu.sync_copy(src_hbm.at[pl.ds(chunk0, HALF)], srcv)
            pltpu.sync_copy(dst_hbm.at[pl.ds(chunk0, HALF)], dstv)
            pltpu.async_copy(hk_hbm.at[srcv.at[0]], rows0, sem0)

            def body(t, carry):
                for b, (rows, sem) in enumerate(bufs):
                    i = t * 2 + b
                    nrows, nsem = bufs[1 - b]
                    pltpu.make_async_copy(
                        hk_hbm.at[srcv.at[i]], rows, sem).wait()

                    @pl.when(i + 1 < HALF)
                    def _():
                        pltpu.async_copy(hk_hbm.at[srcv.at[i + 1]], nrows, nsem)

                    pltpu.sync_copy(rows, acc.at[dstv.at[i]], add=True)
                return carry

            lax.fori_loop(0, HALF // 2, body, 0)
        plsc.subcore_barrier()
        pltpu.sync_copy(
            acc.at[pl.ds(base_row, ROWS_PER_TILE)],
            out_hbm.at[cid, pl.ds(base_row, ROWS_PER_TILE)],
        )

    return sc_degree, sc_aggregate


BLK = 400


def _maxk_rows(h):
    work = h
    t = None
    for _ in range(K):
        t = jnp.max(work, axis=1, keepdims=True)
        work = jnp.where(work >= t, -jnp.inf, work)
    return jnp.where(h >= t, h, 0.0)


def _k0_body(x_ref, w_ref, b_ref, o_ref):
    h = jnp.dot(x_ref[...], w_ref[...], preferred_element_type=jnp.float32)
    o_ref[...] = _maxk_rows(h + b_ref[...])


def _inv_deg(d0_ref, d1_ref):
    deg = d0_ref[...][:, 0:1] + d1_ref[...][:, 0:1]
    return 1.0 / jnp.maximum(deg, 1.0)


def _k1_body(hk_ref, p0_ref, p1_ref, d0_ref, d1_ref,
             ws_ref, wn_ref, b_ref, o_ref):
    agg = (p0_ref[...] + p1_ref[...]) * _inv_deg(d0_ref, d1_ref)
    h = (jnp.dot(hk_ref[...], ws_ref[...], preferred_element_type=jnp.float32)
         + jnp.dot(agg, wn_ref[...], preferred_element_type=jnp.float32)
         + b_ref[...])
    o_ref[...] = _maxk_rows(h)


def _k2_body(hk_ref, p0_ref, p1_ref, d0_ref, d1_ref,
             ws_ref, wn_ref, b_ref, wo_ref, bo_ref, o_ref):
    agg = (p0_ref[...] + p1_ref[...]) * _inv_deg(d0_ref, d1_ref)
    h = (jnp.dot(hk_ref[...], ws_ref[...], preferred_element_type=jnp.float32)
         + jnp.dot(agg, wn_ref[...], preferred_element_type=jnp.float32)
         + b_ref[...])
    o_ref[...] = (jnp.dot(h, wo_ref[...], preferred_element_type=jnp.float32)
                  + bo_ref[...])


def _row_spec():
    return pl.BlockSpec((BLK, D), lambda i: (i, 0))


def _full_spec(shape):
    return pl.BlockSpec(shape, lambda i: tuple(0 for _ in shape))


def _tc_call(body, num_inputs_rowwise, num_full, full_shapes):
    in_specs = [_row_spec() for _ in range(num_inputs_rowwise)]
    in_specs += [_full_spec(s) for s in full_shapes]
    return pl.pallas_call(
        body,
        grid=(N // BLK,),
        in_specs=in_specs,
        out_specs=_row_spec(),
        out_shape=jax.ShapeDtypeStruct((N, D), jnp.float32),
    )


def kernel(x, edge_index, W_in, b_in, W_self, W_neigh, b_neigh, W_out, b_out):
    idx_pad = jnp.full((EPAD - E,), N, jnp.int32)
    src = jnp.concatenate([edge_index[0].astype(jnp.int32), idx_pad])
    src = src.reshape(NCHUNK, CHUNK)
    dst = jnp.concatenate([edge_index[1].astype(jnp.int32), idx_pad])
    dst = dst.reshape(NCHUNK, CHUNK)
    hk_pad = jnp.zeros((NP - N, D), jnp.float32)

    ones_deg = jnp.ones((CHUNK, DEGW), jnp.float32)
    zeros_deg = jnp.zeros((ROWS_PER_TILE, DEGW), jnp.float32)
    zeros_agg = jnp.zeros((ROWS_PER_TILE, D), jnp.float32)

    sc_degree, sc_aggregate = _sc_kernels()
    degp = sc_degree(dst, ones_deg, zeros_deg)
    d0, d1 = degp[0, :N], degp[1, :N]

    b_in2 = b_in.reshape(1, D)
    bo2 = b_out.reshape(1, D)

    k0 = pl.pallas_call(
        _k0_body,
        grid=(N // BLK,),
        in_specs=[_row_spec(), _full_spec((D, D)), _full_spec((1, D))],
        out_specs=_row_spec(),
        out_shape=jax.ShapeDtypeStruct((N, D), jnp.float32),
    )
    hk = k0(x, W_in, b_in2)

    deg_spec = pl.BlockSpec((BLK, DEGW), lambda i: (i, 0))

    k1 = pl.pallas_call(
        _k1_body,
        grid=(N // BLK,),
        in_specs=[_row_spec(), _row_spec(), _row_spec(), deg_spec, deg_spec,
                  _full_spec((D, D)), _full_spec((D, D)), _full_spec((1, D))],
        out_specs=_row_spec(),
        out_shape=jax.ShapeDtypeStruct((N, D), jnp.float32),
    )
    k2 = pl.pallas_call(
        _k2_body,
        grid=(N // BLK,),
        in_specs=[_row_spec(), _row_spec(), _row_spec(), deg_spec, deg_spec,
                  _full_spec((D, D)), _full_spec((D, D)), _full_spec((1, D)),
                  _full_spec((D, D)), _full_spec((1, D))],
        out_specs=_row_spec(),
        out_shape=jax.ShapeDtypeStruct((N, D), jnp.float32),
    )

    for l in range(L):
        hk_p = jnp.concatenate([hk, hk_pad])
        p = sc_aggregate(hk_p, src, dst, zeros_agg)
        p0, p1 = p[0, :N], p[1, :N]
        bl = b_neigh[l].reshape(1, D)
        if l < L - 1:
            hk = k1(hk, p0, p1, d0, d1, W_self[l], W_neigh[l], bl)
        else:
            out = k2(hk, p0, p1, d0, d1, W_self[l], W_neigh[l], bl,
                     W_out, bo2)
    return out


# speedup vs baseline: 2.5888x; 2.5888x over previous
import functools

import jax
import jax.numpy as jnp
from jax import lax
from jax.experimental import pallas as pl
from jax.experimental.pallas import tpu as pltpu
from jax.experimental.pallas import tpu_sc as plsc

N = 10000
E = 320000
D = 128
K = 32
L = 3

NC = 2
NS = 16
NW = NC * NS
CHUNK = 128
CPW = 80
HALF = CPW // 2
NCHUNK = NW * CPW
EPAD = NCHUNK * CHUNK
NP = 10240
ROWS_PER_TILE = NP // NS
DEGW = 128


@functools.lru_cache(maxsize=1)
def _sc_kernels():
    mesh = plsc.VectorSubcoreMesh(
        core_axis_name="c", subcore_axis_name="s", num_cores=NC)

    @functools.partial(
        pl.kernel,
        mesh=mesh,
        out_type=jax.ShapeDtypeStruct((NC, NP, DEGW), jnp.float32),
        scratch_types=[
            pltpu.VMEM((CPW, CHUNK), jnp.int32),
            pltpu.VMEM((CHUNK, DEGW), jnp.float32),
            pltpu.VMEM_SHARED((NP, DEGW), jnp.float32),
        ],
    )
    def sc_degree(dst_hbm, ones_hbm, zeros_hbm, out_hbm, dstv, onesv, acc):
        cid = lax.axis_index("c")
        sid = lax.axis_index("s")
        wid = sid * NC + cid
        base_row = sid * ROWS_PER_TILE
        pltpu.sync_copy(zeros_hbm, acc.at[pl.ds(base_row, ROWS_PER_TILE)])
        pltpu.sync_copy(ones_hbm, onesv)
        pltpu.sync_copy(dst_hbm.at[pl.ds(wid * CPW, CPW)], dstv)
        plsc.subcore_barrier()

        def body(i, carry):
            pltpu.sync_copy(onesv, acc.at[dstv.at[i]], add=True)
            return carry

        lax.fori_loop(0, CPW, body, 0)
        plsc.subcore_barrier()
        pltpu.sync_copy(
            acc.at[pl.ds(base_row, ROWS_PER_TILE)],
            out_hbm.at[cid, pl.ds(base_row, ROWS_PER_TILE)],
        )

    @functools.partial(
        pl.kernel,
        mesh=mesh,
        out_type=jax.ShapeDtypeStruct((NC, NP, D), jnp.float32),
        scratch_types=[
            pltpu.VMEM((HALF, CHUNK), jnp.int32),
            pltpu.VMEM((HALF, CHUNK), jnp.int32),
            pltpu.VMEM((CHUNK, D), jnp.float32),
            pltpu.VMEM((CHUNK, D), jnp.float32),
            pltpu.VMEM_SHARED((NP, D), jnp.float32),
            pltpu.SemaphoreType.DMA,
            pltpu.SemaphoreType.DMA,
        ],
    )
    def sc_aggregate(hk_hbm, src_hbm, dst_hbm, zeros_hbm, out_hbm,
                     srcv, dstv, rows0, rows1, acc, sem0, sem1):
        cid = lax.axis_index("c")
        sid = lax.axis_index("s")
        wid = sid * NC + cid
        base_row = sid * ROWS_PER_TILE
        pltpu.sync_copy(zeros_hbm, acc.at[pl.ds(base_row, ROWS_PER_TILE)])
        plsc.subcore_barrier()

        bufs = ((rows0, sem0), (rows1, sem1))
        for half in range(2):
            chunk0 = wid * CPW + half * HALF
            pltpu.sync_copy(src_hbm.at[pl.ds(chunk0, HALF)], srcv)
            pltpu.sync_copy(dst_hbm.at[pl.ds(chunk0, HALF)], dstv)
            pltpu.async_copy(hk_hbm.at[srcv.at[0]], rows0, sem0)

            def body(t, carry):
                for b, (rows, sem) in enumerate(bufs):
                    i = t * 2 + b
                    nrows, nsem = bufs[1 - b]
                    pltpu.make_async_copy(
                        hk_hbm.at[srcv.at[i]], rows, sem).wait()

                    @pl.when(i + 1 < HALF)
                    def _():
                        pltpu.async_copy(hk_hbm.at[srcv.at[i + 1]], nrows, nsem)

                    pltpu.sync_copy(rows, acc.at[dstv.at[i]], add=True)
                return carry

            lax.fori_loop(0, HALF // 2, body, 0)
        plsc.subcore_barrier()
        pltpu.sync_copy(
            acc.at[pl.ds(base_row, ROWS_PER_TILE)],
            out_hbm.at[cid, pl.ds(base_row, ROWS_PER_TILE)],
        )

    return sc_degree, sc_aggregate


BLK = 400


def _maxk_rows(h):
    work = h
    t = None
    for _ in range(K):
        t = jnp.max(work, axis=1, keepdims=True)
        work = jnp.where(work >= t, -jnp.inf, work)
    return jnp.where(h >= t, h, 0.0)


def _k0_body(x_ref, w_ref, b_ref, o_ref):
    h = jnp.dot(x_ref[...], w_ref[...], preferred_element_type=jnp.float32)
    o_ref[...] = _maxk_rows(h + b_ref[...])


def _inv_deg(d0_ref, d1_ref):
    deg = d0_ref[...][:, 0:1] + d1_ref[...][:, 0:1]
    return 1.0 / jnp.maximum(deg, 1.0)


def _k1_body(hk_ref, p0_ref, p1_ref, d0_ref, d1_ref,
             ws_ref, wn_ref, b_ref, o_ref):
    agg = (p0_ref[...] + p1_ref[...]) * _inv_deg(d0_ref, d1_ref)
    h = (jnp.dot(hk_ref[...], ws_ref[...], preferred_element_type=jnp.float32)
         + jnp.dot(agg, wn_ref[...], preferred_element_type=jnp.float32)
         + b_ref[...])
    o_ref[...] = _maxk_rows(h)


def _k2_body(hk_ref, p0_ref, p1_ref, d0_ref, d1_ref,
             ws_ref, wn_ref, b_ref, wo_ref, bo_ref, o_ref):
    agg = (p0_ref[...] + p1_ref[...]) * _inv_deg(d0_ref, d1_ref)
    h = (jnp.dot(hk_ref[...], ws_ref[...], preferred_element_type=jnp.float32)
         + jnp.dot(agg, wn_ref[...], preferred_element_type=jnp.float32)
         + b_ref[...])
    o_ref[...] = (jnp.dot(h, wo_ref[...], preferred_element_type=jnp.float32)
                  + bo_ref[...])


def _row_spec():
    return pl.BlockSpec((BLK, D), lambda i: (i, 0))


def _full_spec(shape):
    return pl.BlockSpec(shape, lambda i: tuple(0 for _ in shape))


def _tc_call(body, num_inputs_rowwise, num_full, full_shapes):
    in_specs = [_row_spec() for _ in range(num_inputs_rowwise)]
    in_specs += [_full_spec(s) for s in full_shapes]
    return pl.pallas_call(
        body,
        grid=(N // BLK,),
        in_specs=in_specs,
        out_specs=_row_spec(),
        out_shape=jax.ShapeDtypeStruct((N, D), jnp.float32),
    )


def kernel(x, edge_index, W_in, b_in, W_self, W_neigh, b_neigh, W_out, b_out):
    idx_pad = N + jnp.arange(EPAD - E, dtype=jnp.int32) % (NP - N)
    src = jnp.concatenate([edge_index[0].astype(jnp.int32), idx_pad])
    src = src.reshape(NCHUNK, CHUNK)
    dst = jnp.concatenate([edge_index[1].astype(jnp.int32), idx_pad])
    dst = dst.reshape(NCHUNK, CHUNK)
    hk_pad = jnp.zeros((NP - N, D), jnp.float32)

    ones_deg = jnp.ones((CHUNK, DEGW), jnp.float32)
    zeros_deg = jnp.zeros((ROWS_PER_TILE, DEGW), jnp.float32)
    zeros_agg = jnp.zeros((ROWS_PER_TILE, D), jnp.float32)

    sc_degree, sc_aggregate = _sc_kernels()
    degp = sc_degree(dst, ones_deg, zeros_deg)
    d0, d1 = degp[0, :N], degp[1, :N]

    b_in2 = b_in.reshape(1, D)
    bo2 = b_out.reshape(1, D)

    k0 = pl.pallas_call(
        _k0_body,
        grid=(N // BLK,),
        in_specs=[_row_spec(), _full_spec((D, D)), _full_spec((1, D))],
        out_specs=_row_spec(),
        out_shape=jax.ShapeDtypeStruct((N, D), jnp.float32),
    )
    hk = k0(x, W_in, b_in2)

    deg_spec = pl.BlockSpec((BLK, DEGW), lambda i: (i, 0))

    k1 = pl.pallas_call(
        _k1_body,
        grid=(N // BLK,),
        in_specs=[_row_spec(), _row_spec(), _row_spec(), deg_spec, deg_spec,
                  _full_spec((D, D)), _full_spec((D, D)), _full_spec((1, D))],
        out_specs=_row_spec(),
        out_shape=jax.ShapeDtypeStruct((N, D), jnp.float32),
    )
    k2 = pl.pallas_call(
        _k2_body,
        grid=(N // BLK,),
        in_specs=[_row_spec(), _row_spec(), _row_spec(), deg_spec, deg_spec,
                  _full_spec((D, D)), _full_spec((D, D)), _full_spec((1, D)),
                  _full_spec((D, D)), _full_spec((1, D))],
        out_specs=_row_spec(),
        out_shape=jax.ShapeDtypeStruct((N, D), jnp.float32),
    )

    for l in range(L):
        hk_p = jnp.concatenate([hk, hk_pad])
        p = sc_aggregate(hk_p, src, dst, zeros_agg)
        p0, p1 = p[0, :N], p[1, :N]
        bl = b_neigh[l].reshape(1, D)
        if l < L - 1:
            hk = k1(hk, p0, p1, d0, d1, W_self[l], W_neigh[l], bl)
        else:
            out = k2(hk, p0, p1, d0, d1, W_self[l], W_neigh[l], bl,
                     W_out, bo2)
    return out


# speedup vs baseline: 2.8637x; 1.1062x over previous
import functools

import jax
import jax.numpy as jnp
from jax import lax
from jax.experimental import pallas as pl
from jax.experimental.pallas import tpu as pltpu
from jax.experimental.pallas import tpu_sc as plsc

N = 10000
E = 320000
D = 128
K = 32
L = 3

NC = 2
NS = 16
NW = NC * NS
CHUNK = 128
CPW = 80
HALF = CPW // 2
NCHUNK = NW * CPW
EPAD = NCHUNK * CHUNK
NP = 10240
ROWS_PER_TILE = NP // NS
DEGW = 128


@functools.lru_cache(maxsize=1)
def _sc_kernels():
    mesh = plsc.VectorSubcoreMesh(
        core_axis_name="c", subcore_axis_name="s", num_cores=NC)

    @functools.partial(
        pl.kernel,
        mesh=mesh,
        out_type=jax.ShapeDtypeStruct((NC, NP, DEGW), jnp.float32),
        scratch_types=[
            pltpu.VMEM((CPW, CHUNK), jnp.int32),
            pltpu.VMEM((CHUNK, DEGW), jnp.float32),
            pltpu.VMEM_SHARED((NP, DEGW), jnp.float32),
        ],
    )
    def sc_degree(dst_hbm, ones_hbm, zeros_hbm, out_hbm, dstv, onesv, acc):
        cid = lax.axis_index("c")
        sid = lax.axis_index("s")
        wid = sid * NC + cid
        base_row = sid * ROWS_PER_TILE
        pltpu.sync_copy(zeros_hbm, acc.at[pl.ds(base_row, ROWS_PER_TILE)])
        pltpu.sync_copy(ones_hbm, onesv)
        pltpu.sync_copy(dst_hbm.at[pl.ds(wid * CPW, CPW)], dstv)
        plsc.subcore_barrier()

        def body(i, carry):
            pltpu.sync_copy(onesv, acc.at[dstv.at[i]], add=True)
            return carry

        lax.fori_loop(0, CPW, body, 0)
        plsc.subcore_barrier()
        pltpu.sync_copy(
            acc.at[pl.ds(base_row, ROWS_PER_TILE)],
            out_hbm.at[cid, pl.ds(base_row, ROWS_PER_TILE)],
        )

    @functools.partial(
        pl.kernel,
        mesh=mesh,
        out_type=jax.ShapeDtypeStruct((NC, NP, D), jnp.float32),
        scratch_types=[
            pltpu.VMEM((HALF, CHUNK), jnp.int32),
            pltpu.VMEM((HALF, CHUNK), jnp.int32),
            pltpu.VMEM((CHUNK, D), jnp.float32),
            pltpu.VMEM((CHUNK, D), jnp.float32),
            pltpu.VMEM_SHARED((NP, D), jnp.float32),
            pltpu.SemaphoreType.DMA,
            pltpu.SemaphoreType.DMA,
        ],
    )
    def sc_aggregate(hk_hbm, src_hbm, dst_hbm, zeros_hbm, out_hbm,
                     srcv, dstv, rows0, rows1, acc, sem0, sem1):
        cid = lax.axis_index("c")
        sid = lax.axis_index("s")
        wid = sid * NC + cid
        base_row = sid * ROWS_PER_TILE
        pltpu.sync_copy(zeros_hbm, acc.at[pl.ds(base_row, ROWS_PER_TILE)])
        plsc.subcore_barrier()

        bufs = ((rows0, sem0), (rows1, sem1))
        for half in range(2):
            chunk0 = wid * CPW + half * HALF
            pltpu.sync_copy(src_hbm.at[pl.ds(chunk0, HALF)], srcv)
            pltpu.sync_copy(dst_hbm.at[pl.ds(chunk0, HALF)], dstv)
            pltpu.async_copy(hk_hbm.at[srcv.at[0]], rows0, sem0)

            def body(t, carry):
                for b, (rows, sem) in enumerate(bufs):
                    i = t * 2 + b
                    nrows, nsem = bufs[1 - b]
                    pltpu.make_async_copy(
                        hk_hbm.at[srcv.at[i]], rows, sem).wait()

                    @pl.when(i + 1 < HALF)
                    def _():
                        pltpu.async_copy(hk_hbm.at[srcv.at[i + 1]], nrows, nsem)

                    pltpu.sync_copy(rows, acc.at[dstv.at[i]], add=True)
                return carry

            lax.fori_loop(0, HALF // 2, body, 0)
        plsc.subcore_barrier()
        pltpu.sync_copy(
            acc.at[pl.ds(base_row, ROWS_PER_TILE)],
            out_hbm.at[cid, pl.ds(base_row, ROWS_PER_TILE)],
        )

    return sc_degree, sc_aggregate


BLK = 1000


def _maxk_rows(h):
    work = h
    t = None
    for _ in range(K):
        t = jnp.max(work, axis=1, keepdims=True)
        work = jnp.where(work >= t, -jnp.inf, work)
    return jnp.where(h >= t, h, 0.0)


def _k0_body(x_ref, w_ref, b_ref, o_ref):
    h = jnp.dot(x_ref[...], w_ref[...], preferred_element_type=jnp.float32)
    o_ref[...] = _maxk_rows(h + b_ref[...])


def _inv_deg(d0_ref, d1_ref):
    deg = d0_ref[...][:, 0:1] + d1_ref[...][:, 0:1]
    return 1.0 / jnp.maximum(deg, 1.0)


def _k1_body(hk_ref, p0_ref, p1_ref, d0_ref, d1_ref,
             ws_ref, wn_ref, b_ref, o_ref):
    agg = (p0_ref[...] + p1_ref[...]) * _inv_deg(d0_ref, d1_ref)
    h = (jnp.dot(hk_ref[...], ws_ref[...], preferred_element_type=jnp.float32)
         + jnp.dot(agg, wn_ref[...], preferred_element_type=jnp.float32)
         + b_ref[...])
    o_ref[...] = _maxk_rows(h)


def _k2_body(hk_ref, p0_ref, p1_ref, d0_ref, d1_ref,
             ws_ref, wn_ref, b_ref, wo_ref, bo_ref, o_ref):
    agg = (p0_ref[...] + p1_ref[...]) * _inv_deg(d0_ref, d1_ref)
    h = (jnp.dot(hk_ref[...], ws_ref[...], preferred_element_type=jnp.float32)
         + jnp.dot(agg, wn_ref[...], preferred_element_type=jnp.float32)
         + b_ref[...])
    o_ref[...] = (jnp.dot(h, wo_ref[...], preferred_element_type=jnp.float32)
                  + bo_ref[...])


def _row_spec():
    return pl.BlockSpec((BLK, D), lambda i: (i, 0))


def _full_spec(shape):
    return pl.BlockSpec(shape, lambda i: tuple(0 for _ in shape))


def _tc_call(body, num_inputs_rowwise, num_full, full_shapes):
    in_specs = [_row_spec() for _ in range(num_inputs_rowwise)]
    in_specs += [_full_spec(s) for s in full_shapes]
    return pl.pallas_call(
        body,
        grid=(N // BLK,),
        in_specs=in_specs,
        out_specs=_row_spec(),
        out_shape=jax.ShapeDtypeStruct((N, D), jnp.float32),
    )


def kernel(x, edge_index, W_in, b_in, W_self, W_neigh, b_neigh, W_out, b_out):
    idx_pad = N + jnp.arange(EPAD - E, dtype=jnp.int32) % (NP - N)
    src = jnp.concatenate([edge_index[0].astype(jnp.int32), idx_pad])
    src = src.reshape(NCHUNK, CHUNK)
    dst = jnp.concatenate([edge_index[1].astype(jnp.int32), idx_pad])
    dst = dst.reshape(NCHUNK, CHUNK)
    hk_pad = jnp.zeros((NP - N, D), jnp.float32)

    ones_deg = jnp.ones((CHUNK, DEGW), jnp.float32)
    zeros_deg = jnp.zeros((ROWS_PER_TILE, DEGW), jnp.float32)
    zeros_agg = jnp.zeros((ROWS_PER_TILE, D), jnp.float32)

    sc_degree, sc_aggregate = _sc_kernels()
    degp = sc_degree(dst, ones_deg, zeros_deg)
    d0, d1 = degp[0, :N], degp[1, :N]

    b_in2 = b_in.reshape(1, D)
    bo2 = b_out.reshape(1, D)

    k0 = pl.pallas_call(
        _k0_body,
        grid=(N // BLK,),
        in_specs=[_row_spec(), _full_spec((D, D)), _full_spec((1, D))],
        out_specs=_row_spec(),
        out_shape=jax.ShapeDtypeStruct((N, D), jnp.float32),
    )
    hk = k0(x, W_in, b_in2)

    deg_spec = pl.BlockSpec((BLK, DEGW), lambda i: (i, 0))

    k1 = pl.pallas_call(
        _k1_body,
        grid=(N // BLK,),
        in_specs=[_row_spec(), _row_spec(), _row_spec(), deg_spec, deg_spec,
                  _full_spec((D, D)), _full_spec((D, D)), _full_spec((1, D))],
        out_specs=_row_spec(),
        out_shape=jax.ShapeDtypeStruct((N, D), jnp.float32),
    )
    k2 = pl.pallas_call(
        _k2_body,
        grid=(N // BLK,),
        in_specs=[_row_spec(), _row_spec(), _row_spec(), deg_spec, deg_spec,
                  _full_spec((D, D)), _full_spec((D, D)), _full_spec((1, D)),
                  _full_spec((D, D)), _full_spec((1, D))],
        out_specs=_row_spec(),
        out_shape=jax.ShapeDtypeStruct((N, D), jnp.float32),
    )

    for l in range(L):
        hk_p = jnp.concatenate([hk, hk_pad])
        p = sc_aggregate(hk_p, src, dst, zeros_agg)
        p0, p1 = p[0, :N], p[1, :N]
        bl = b_neigh[l].reshape(1, D)
        if l < L - 1:
            hk = k1(hk, p0, p1, d0, d1, W_self[l], W_neigh[l], bl)
        else:
            out = k2(hk, p0, p1, d0, d1, W_self[l], W_neigh[l], bl,
                     W_out, bo2)
    return out


# speedup vs baseline: 2.8769x; 1.0046x over previous
import functools

import jax
import jax.numpy as jnp
from jax import lax
from jax.experimental import pallas as pl
from jax.experimental.pallas import tpu as pltpu
from jax.experimental.pallas import tpu_sc as plsc

N = 10000
E = 320000
D = 128
K = 32
L = 3

NC = 2
NS = 16
NW = NC * NS
CHUNK = 128
CPW = 80
HALF = CPW // 2
NCHUNK = NW * CPW
EPAD = NCHUNK * CHUNK
NP = 10240
ROWS_PER_TILE = NP // NS
DEGW = 128


@functools.lru_cache(maxsize=1)
def _sc_kernels():
    mesh = plsc.VectorSubcoreMesh(
        core_axis_name="c", subcore_axis_name="s", num_cores=NC)

    @functools.partial(
        pl.kernel,
        mesh=mesh,
        out_type=jax.ShapeDtypeStruct((NC, NP, DEGW), jnp.float32),
        scratch_types=[
            pltpu.VMEM((CPW, CHUNK), jnp.int32),
            pltpu.VMEM((CHUNK, DEGW), jnp.float32),
            pltpu.VMEM_SHARED((NP, DEGW), jnp.float32),
        ],
    )
    def sc_degree(dst_hbm, ones_hbm, zeros_hbm, out_hbm, dstv, onesv, acc):
        cid = lax.axis_index("c")
        sid = lax.axis_index("s")
        wid = sid * NC + cid
        base_row = sid * ROWS_PER_TILE
        pltpu.sync_copy(zeros_hbm, acc.at[pl.ds(base_row, ROWS_PER_TILE)])
        pltpu.sync_copy(ones_hbm, onesv)
        pltpu.sync_copy(dst_hbm.at[pl.ds(wid * CPW, CPW)], dstv)
        plsc.subcore_barrier()

        def body(i, carry):
            pltpu.sync_copy(onesv, acc.at[dstv.at[i]], add=True)
            return carry

        lax.fori_loop(0, CPW, body, 0)
        plsc.subcore_barrier()
        pltpu.sync_copy(
            acc.at[pl.ds(base_row, ROWS_PER_TILE)],
            out_hbm.at[cid, pl.ds(base_row, ROWS_PER_TILE)],
        )

    @functools.partial(
        pl.kernel,
        mesh=mesh,
        out_type=jax.ShapeDtypeStruct((NC, NP, D), jnp.float32),
        scratch_types=[
            pltpu.VMEM((HALF, CHUNK), jnp.int32),
            pltpu.VMEM((HALF, CHUNK), jnp.int32),
            pltpu.VMEM((CHUNK, D), jnp.float32),
            pltpu.VMEM((CHUNK, D), jnp.float32),
            pltpu.VMEM_SHARED((NP, D), jnp.float32),
            pltpu.SemaphoreType.DMA,
            pltpu.SemaphoreType.DMA,
        ],
    )
    def sc_aggregate(hk_hbm, src_hbm, dst_hbm, zeros_hbm, out_hbm,
                     srcv, dstv, rows0, rows1, acc, sem0, sem1):
        cid = lax.axis_index("c")
        sid = lax.axis_index("s")
        wid = sid * NC + cid
        base_row = sid * ROWS_PER_TILE
        pltpu.sync_copy(zeros_hbm, acc.at[pl.ds(base_row, ROWS_PER_TILE)])
        plsc.subcore_barrier()

        bufs = ((rows0, sem0), (rows1, sem1))
        for half in range(2):
            chunk0 = wid * CPW + half * HALF
            pltpu.sync_copy(src_hbm.at[pl.ds(chunk0, HALF)], srcv)
            pltpu.sync_copy(dst_hbm.at[pl.ds(chunk0, HALF)], dstv)
            pltpu.async_copy(hk_hbm.at[srcv.at[0]], rows0, sem0)

            def body(t, carry):
                for b, (rows, sem) in enumerate(bufs):
                    i = t * 2 + b
                    nrows, nsem = bufs[1 - b]
                    pltpu.make_async_copy(
                        hk_hbm.at[srcv.at[i]], rows, sem).wait()

                    @pl.when(i + 1 < HALF)
                    def _():
                        pltpu.async_copy(hk_hbm.at[srcv.at[i + 1]], nrows, nsem)

                    pltpu.sync_copy(rows, acc.at[dstv.at[i]], add=True)
                return carry

            lax.fori_loop(0, HALF // 2, body, 0)
        plsc.subcore_barrier()
        pltpu.sync_copy(
            acc.at[pl.ds(base_row, ROWS_PER_TILE)],
            out_hbm.at[cid, pl.ds(base_row, ROWS_PER_TILE)],
        )

    return sc_degree, sc_aggregate


BLK = 2000


def _maxk_rows(h):
    work = h
    t = None
    for _ in range(K):
        t = jnp.max(work, axis=1, keepdims=True)
        work = jnp.where(work >= t, -jnp.inf, work)
    return jnp.where(h >= t, h, 0.0)


def _k0_body(x_ref, w_ref, b_ref, o_ref):
    h = jnp.dot(x_ref[...], w_ref[...], preferred_element_type=jnp.float32)
    o_ref[...] = _maxk_rows(h + b_ref[...])


def _inv_deg(d0_ref, d1_ref):
    deg = d0_ref[...][:, 0:1] + d1_ref[...][:, 0:1]
    return 1.0 / jnp.maximum(deg, 1.0)


def _k1_body(hk_ref, p0_ref, p1_ref, d0_ref, d1_ref,
             ws_ref, wn_ref, b_ref, o_ref):
    agg = (p0_ref[...] + p1_ref[...]) * _inv_deg(d0_ref, d1_ref)
    h = (jnp.dot(hk_ref[...], ws_ref[...], preferred_element_type=jnp.float32)
         + jnp.dot(agg, wn_ref[...], preferred_element_type=jnp.float32)
         + b_ref[...])
    o_ref[...] = _maxk_rows(h)


def _k2_body(hk_ref, p0_ref, p1_ref, d0_ref, d1_ref,
             ws_ref, wn_ref, b_ref, wo_ref, bo_ref, o_ref):
    agg = (p0_ref[...] + p1_ref[...]) * _inv_deg(d0_ref, d1_ref)
    h = (jnp.dot(hk_ref[...], ws_ref[...], preferred_element_type=jnp.float32)
         + jnp.dot(agg, wn_ref[...], preferred_element_type=jnp.float32)
         + b_ref[...])
    o_ref[...] = (jnp.dot(h, wo_ref[...], preferred_element_type=jnp.float32)
                  + bo_ref[...])


def _row_spec():
    return pl.BlockSpec((BLK, D), lambda i: (i, 0))


def _full_spec(shape):
    return pl.BlockSpec(shape, lambda i: tuple(0 for _ in shape))


def _tc_call(body, num_inputs_rowwise, num_full, full_shapes):
    in_specs = [_row_spec() for _ in range(num_inputs_rowwise)]
    in_specs += [_full_spec(s) for s in full_shapes]
    return pl.pallas_call(
        body,
        grid=(N // BLK,),
        in_specs=in_specs,
        out_specs=_row_spec(),
        out_shape=jax.ShapeDtypeStruct((N, D), jnp.float32),
    )


def kernel(x, edge_index, W_in, b_in, W_self, W_neigh, b_neigh, W_out, b_out):
    idx_pad = N + jnp.arange(EPAD - E, dtype=jnp.int32) % (NP - N)
    src = jnp.concatenate([edge_index[0].astype(jnp.int32), idx_pad])
    src = src.reshape(NCHUNK, CHUNK)
    dst = jnp.concatenate([edge_index[1].astype(jnp.int32), idx_pad])
    dst = dst.reshape(NCHUNK, CHUNK)
    hk_pad = jnp.zeros((NP - N, D), jnp.float32)

    ones_deg = jnp.ones((CHUNK, DEGW), jnp.float32)
    zeros_deg = jnp.zeros((ROWS_PER_TILE, DEGW), jnp.float32)
    zeros_agg = jnp.zeros((ROWS_PER_TILE, D), jnp.float32)

    sc_degree, sc_aggregate = _sc_kernels()
    degp = sc_degree(dst, ones_deg, zeros_deg)
    d0, d1 = degp[0, :N], degp[1, :N]

    b_in2 = b_in.reshape(1, D)
    bo2 = b_out.reshape(1, D)

    k0 = pl.pallas_call(
        _k0_body,
        grid=(N // BLK,),
        in_specs=[_row_spec(), _full_spec((D, D)), _full_spec((1, D))],
        out_specs=_row_spec(),
        out_shape=jax.ShapeDtypeStruct((N, D), jnp.float32),
    )
    hk = k0(x, W_in, b_in2)

    deg_spec = pl.BlockSpec((BLK, DEGW), lambda i: (i, 0))

    k1 = pl.pallas_call(
        _k1_body,
        grid=(N // BLK,),
        in_specs=[_row_spec(), _row_spec(), _row_spec(), deg_spec, deg_spec,
                  _full_spec((D, D)), _full_spec((D, D)), _full_spec((1, D))],
        out_specs=_row_spec(),
        out_shape=jax.ShapeDtypeStruct((N, D), jnp.float32),
    )
    k2 = pl.pallas_call(
        _k2_body,
        grid=(N // BLK,),
        in_specs=[_row_spec(), _row_spec(), _row_spec(), deg_spec, deg_spec,
                  _full_spec((D, D)), _full_spec((D, D)), _full_spec((1, D)),
                  _full_spec((D, D)), _full_spec((1, D))],
        out_specs=_row_spec(),
        out_shape=jax.ShapeDtypeStruct((N, D), jnp.float32),
    )

    for l in range(L):
        hk_p = jnp.concatenate([hk, hk_pad])
        p = sc_aggregate(hk_p, src, dst, zeros_agg)
        p0, p1 = p[0, :N], p[1, :N]
        bl = b_neigh[l].reshape(1, D)
        if l < L - 1:
            hk = k1(hk, p0, p1, d0, d1, W_self[l], W_neigh[l], bl)
        else:
            out = k2(hk, p0, p1, d0, d1, W_self[l], W_neigh[l], bl,
                     W_out, bo2)
    return out


# speedup vs baseline: 2.8769x; 1.0000x over previous
import functools

import jax
import jax.numpy as jnp
from jax import lax
from jax.experimental import pallas as pl
from jax.experimental.pallas import tpu as pltpu
from jax.experimental.pallas import tpu_sc as plsc

N = 10000
E = 320000
D = 128
K = 32
L = 3

NC = 2
NS = 16
NW = NC * NS
CHUNK = 128
CPW = 80
HALF = CPW // 2
NCHUNK = NW * CPW
EPAD = NCHUNK * CHUNK
NP = 10240
ROWS_PER_TILE = NP // NS
DEGW = 128


@functools.lru_cache(maxsize=1)
def _sc_kernels():
    mesh = plsc.VectorSubcoreMesh(
        core_axis_name="c", subcore_axis_name="s", num_cores=NC)

    @functools.partial(
        pl.kernel,
        mesh=mesh,
        out_type=jax.ShapeDtypeStruct((NC, NP, DEGW), jnp.float32),
        scratch_types=[
            pltpu.VMEM((CPW, CHUNK), jnp.int32),
            pltpu.VMEM((CHUNK, DEGW), jnp.float32),
            pltpu.VMEM_SHARED((NP, DEGW), jnp.float32),
        ],
    )
    def sc_degree(dst_hbm, ones_hbm, zeros_hbm, out_hbm, dstv, onesv, acc):
        cid = lax.axis_index("c")
        sid = lax.axis_index("s")
        wid = sid * NC + cid
        base_row = sid * ROWS_PER_TILE
        pltpu.sync_copy(zeros_hbm, acc.at[pl.ds(base_row, ROWS_PER_TILE)])
        pltpu.sync_copy(ones_hbm, onesv)
        pltpu.sync_copy(dst_hbm.at[pl.ds(wid * CPW, CPW)], dstv)
        plsc.subcore_barrier()

        def body(i, carry):
            pltpu.sync_copy(onesv, acc.at[dstv.at[i]], add=True)
            return carry

        lax.fori_loop(0, CPW, body, 0)
        plsc.subcore_barrier()
        pltpu.sync_copy(
            acc.at[pl.ds(base_row, ROWS_PER_TILE)],
            out_hbm.at[cid, pl.ds(base_row, ROWS_PER_TILE)],
        )

    @functools.partial(
        pl.kernel,
        mesh=mesh,
        out_type=jax.ShapeDtypeStruct((NC, NP, D), jnp.float32),
        scratch_types=[
            pltpu.VMEM((HALF, CHUNK), jnp.int32),
            pltpu.VMEM((HALF, CHUNK), jnp.int32),
            pltpu.VMEM((CHUNK, D), jnp.float32),
            pltpu.VMEM((CHUNK, D), jnp.float32),
            pltpu.VMEM_SHARED((NP, D), jnp.float32),
            pltpu.SemaphoreType.DMA,
            pltpu.SemaphoreType.DMA,
            pltpu.SemaphoreType.DMA,
            pltpu.SemaphoreType.DMA,
        ],
    )
    def sc_aggregate(hk_hbm, src_hbm, dst_hbm, zeros_hbm, out_hbm,
                     srcv, dstv, rows0, rows1, acc,
                     gsem0, gsem1, ssem0, ssem1):
        cid = lax.axis_index("c")
        sid = lax.axis_index("s")
        wid = sid * NC + cid
        base_row = sid * ROWS_PER_TILE
        pltpu.sync_copy(zeros_hbm, acc.at[pl.ds(base_row, ROWS_PER_TILE)])
        plsc.subcore_barrier()

        gbufs = ((rows0, gsem0), (rows1, gsem1))
        ssems = (ssem0, ssem1)
        for half in range(2):
            chunk0 = wid * CPW + half * HALF
            pltpu.sync_copy(src_hbm.at[pl.ds(chunk0, HALF)], srcv)
            pltpu.sync_copy(dst_hbm.at[pl.ds(chunk0, HALF)], dstv)
            pltpu.async_copy(hk_hbm.at[srcv.at[0]], rows0, gsem0)

            def body(t, carry):
                for b, (rows, gsem) in enumerate(gbufs):
                    i = t * 2 + b
                    nrows, ngsem = gbufs[1 - b]
                    ssem, nssem = ssems[b], ssems[1 - b]
                    pltpu.make_async_copy(
                        hk_hbm.at[srcv.at[i]], rows, gsem).wait()
                    pltpu.async_copy(rows, acc.at[dstv.at[i]], ssem, add=True)

                    @pl.when((i + 1 < HALF) & (i >= 1))
                    def _():
                        pltpu.make_async_copy(
                            rows, acc.at[dstv.at[i]], nssem).wait()

                    @pl.when(i + 1 < HALF)
                    def _():
                        pltpu.async_copy(
                            hk_hbm.at[srcv.at[i + 1]], nrows, ngsem)
                return carry

            lax.fori_loop(0, HALF // 2, body, 0)
            pltpu.make_async_copy(rows0, acc.at[dstv.at[0]], ssem0).wait()
            pltpu.make_async_copy(rows1, acc.at[dstv.at[0]], ssem1).wait()
        plsc.subcore_barrier()
        pltpu.sync_copy(
            acc.at[pl.ds(base_row, ROWS_PER_TILE)],
            out_hbm.at[cid, pl.ds(base_row, ROWS_PER_TILE)],
        )

    return sc_degree, sc_aggregate


BLK = 2000


def _maxk_rows(h):
    work = h
    t = None
    for _ in range(K):
        t = jnp.max(work, axis=1, keepdims=True)
        work = jnp.where(work >= t, -jnp.inf, work)
    return jnp.where(h >= t, h, 0.0)


def _k0_body(x_ref, w_ref, b_ref, o_ref):
    h = jnp.dot(x_ref[...], w_ref[...], preferred_element_type=jnp.float32)
    o_ref[...] = _maxk_rows(h + b_ref[...])


def _inv_deg(d0_ref, d1_ref):
    deg = d0_ref[...][:, 0:1] + d1_ref[...][:, 0:1]
    return 1.0 / jnp.maximum(deg, 1.0)


def _k1_body(hk_ref, p0_ref, p1_ref, d0_ref, d1_ref,
             ws_ref, wn_ref, b_ref, o_ref):
    agg = (p0_ref[...] + p1_ref[...]) * _inv_deg(d0_ref, d1_ref)
    h = (jnp.dot(hk_ref[...], ws_ref[...], preferred_element_type=jnp.float32)
         + jnp.dot(agg, wn_ref[...], preferred_element_type=jnp.float32)
         + b_ref[...])
    o_ref[...] = _maxk_rows(h)


def _k2_body(hk_ref, p0_ref, p1_ref, d0_ref, d1_ref,
             ws_ref, wn_ref, b_ref, wo_ref, bo_ref, o_ref):
    agg = (p0_ref[...] + p1_ref[...]) * _inv_deg(d0_ref, d1_ref)
    h = (jnp.dot(hk_ref[...], ws_ref[...], preferred_element_type=jnp.float32)
         + jnp.dot(agg, wn_ref[...], preferred_element_type=jnp.float32)
         + b_ref[...])
    o_ref[...] = (jnp.dot(h, wo_ref[...], preferred_element_type=jnp.float32)
                  + bo_ref[...])


def _row_spec():
    return pl.BlockSpec((BLK, D), lambda i: (i, 0))


def _full_spec(shape):
    return pl.BlockSpec(shape, lambda i: tuple(0 for _ in shape))


def _tc_call(body, num_inputs_rowwise, num_full, full_shapes):
    in_specs = [_row_spec() for _ in range(num_inputs_rowwise)]
    in_specs += [_full_spec(s) for s in full_shapes]
    return pl.pallas_call(
        body,
        grid=(N // BLK,),
        in_specs=in_specs,
        out_specs=_row_spec(),
        out_shape=jax.ShapeDtypeStruct((N, D), jnp.float32),
    )


def kernel(x, edge_index, W_in, b_in, W_self, W_neigh, b_neigh, W_out, b_out):
    idx_pad = N + jnp.arange(EPAD - E, dtype=jnp.int32) % (NP - N)
    src = jnp.concatenate([edge_index[0].astype(jnp.int32), idx_pad])
    src = src.reshape(NCHUNK, CHUNK)
    dst = jnp.concatenate([edge_index[1].astype(jnp.int32), idx_pad])
    dst = dst.reshape(NCHUNK, CHUNK)
    hk_pad = jnp.zeros((NP - N, D), jnp.float32)

    ones_deg = jnp.ones((CHUNK, DEGW), jnp.float32)
    zeros_deg = jnp.zeros((ROWS_PER_TILE, DEGW), jnp.float32)
    zeros_agg = jnp.zeros((ROWS_PER_TILE, D), jnp.float32)

    sc_degree, sc_aggregate = _sc_kernels()
    degp = sc_degree(dst, ones_deg, zeros_deg)
    d0, d1 = degp[0, :N], degp[1, :N]

    b_in2 = b_in.reshape(1, D)
    bo2 = b_out.reshape(1, D)

    k0 = pl.pallas_call(
        _k0_body,
        grid=(N // BLK,),
        in_specs=[_row_spec(), _full_spec((D, D)), _full_spec((1, D))],
        out_specs=_row_spec(),
        out_shape=jax.ShapeDtypeStruct((N, D), jnp.float32),
    )
    hk = k0(x, W_in, b_in2)

    deg_spec = pl.BlockSpec((BLK, DEGW), lambda i: (i, 0))

    k1 = pl.pallas_call(
        _k1_body,
        grid=(N // BLK,),
        in_specs=[_row_spec(), _row_spec(), _row_spec(), deg_spec, deg_spec,
                  _full_spec((D, D)), _full_spec((D, D)), _full_spec((1, D))],
        out_specs=_row_spec(),
        out_shape=jax.ShapeDtypeStruct((N, D), jnp.float32),
    )
    k2 = pl.pallas_call(
        _k2_body,
        grid=(N // BLK,),
        in_specs=[_row_spec(), _row_spec(), _row_spec(), deg_spec, deg_spec,
                  _full_spec((D, D)), _full_spec((D, D)), _full_spec((1, D)),
                  _full_spec((D, D)), _full_spec((1, D))],
        out_specs=_row_spec(),
        out_shape=jax.ShapeDtypeStruct((N, D), jnp.float32),
    )

    for l in range(L):
        p = sc_aggregate(jnp.concatenate([hk, hk_pad]), src, dst, zeros_agg)
        p0, p1 = p[0, :N], p[1, :N]
        bl = b_neigh[l].reshape(1, D)
        if l < L - 1:
            hk = k1(hk, p0, p1, d0, d1, W_self[l], W_neigh[l], bl)
        else:
            out = k2(hk, p0, p1, d0, d1, W_self[l], W_neigh[l], bl,
                     W_out, bo2)
    return out


# speedup vs baseline: 3.1015x; 1.0781x over previous
import functools

import jax
import jax.numpy as jnp
from jax import lax
from jax.experimental import pallas as pl
from jax.experimental.pallas import tpu as pltpu
from jax.experimental.pallas import tpu_sc as plsc

N = 10000
E = 320000
D = 128
K = 32
L = 3

NC = 2
NS = 16
NW = NC * NS
CHUNK = 128
CPW = 80
HALF = CPW // 2
NCHUNK = NW * CPW
EPAD = NCHUNK * CHUNK
NP = 10240
ROWS_PER_TILE = NP // NS
DEGW = 128


@functools.lru_cache(maxsize=1)
def _sc_kernels():
    mesh = plsc.VectorSubcoreMesh(
        core_axis_name="c", subcore_axis_name="s", num_cores=NC)

    @functools.partial(
        pl.kernel,
        mesh=mesh,
        out_type=jax.ShapeDtypeStruct((NC, NP, DEGW), jnp.float32),
        scratch_types=[
            pltpu.VMEM((CPW, CHUNK), jnp.int32),
            pltpu.VMEM((CHUNK, DEGW), jnp.float32),
            pltpu.VMEM_SHARED((NP, DEGW), jnp.float32),
        ],
    )
    def sc_degree(dst_hbm, ones_hbm, zeros_hbm, out_hbm, dstv, onesv, acc):
        cid = lax.axis_index("c")
        sid = lax.axis_index("s")
        wid = sid * NC + cid
        base_row = sid * ROWS_PER_TILE
        pltpu.sync_copy(zeros_hbm, acc.at[pl.ds(base_row, ROWS_PER_TILE)])
        pltpu.sync_copy(ones_hbm, onesv)
        pltpu.sync_copy(dst_hbm.at[pl.ds(wid * CPW, CPW)], dstv)
        plsc.subcore_barrier()

        def body(i, carry):
            pltpu.sync_copy(onesv, acc.at[dstv.at[i]], add=True)
            return carry

        lax.fori_loop(0, CPW, body, 0)
        plsc.subcore_barrier()
        pltpu.sync_copy(
            acc.at[pl.ds(base_row, ROWS_PER_TILE)],
            out_hbm.at[cid, pl.ds(base_row, ROWS_PER_TILE)],
        )

    @functools.partial(
        pl.kernel,
        mesh=mesh,
        out_type=jax.ShapeDtypeStruct((NC, NP, D), jnp.float32),
        scratch_types=[
            pltpu.VMEM((HALF, CHUNK), jnp.int32),
            pltpu.VMEM((HALF, CHUNK), jnp.int32),
            pltpu.VMEM((CHUNK, D), jnp.float32),
            pltpu.VMEM((CHUNK, D), jnp.float32),
            pltpu.VMEM_SHARED((NP, D), jnp.float32),
            pltpu.SemaphoreType.DMA,
            pltpu.SemaphoreType.DMA,
            pltpu.SemaphoreType.DMA,
            pltpu.SemaphoreType.DMA,
        ],
    )
    def sc_aggregate(hk_hbm, src_hbm, dst_hbm, zeros_hbm, out_hbm,
                     srcv, dstv, rows0, rows1, acc,
                     gsem0, gsem1, ssem0, ssem1):
        cid = lax.axis_index("c")
        sid = lax.axis_index("s")
        wid = sid * NC + cid
        base_row = sid * ROWS_PER_TILE
        pltpu.sync_copy(zeros_hbm, acc.at[pl.ds(base_row, ROWS_PER_TILE)])
        plsc.subcore_barrier()

        gbufs = ((rows0, gsem0), (rows1, gsem1))
        ssems = (ssem0, ssem1)
        for half in range(2):
            chunk0 = wid * CPW + half * HALF
            pltpu.sync_copy(src_hbm.at[pl.ds(chunk0, HALF)], srcv)
            pltpu.sync_copy(dst_hbm.at[pl.ds(chunk0, HALF)], dstv)
            pltpu.async_copy(hk_hbm.at[srcv.at[0]], rows0, gsem0)

            def body(t, carry):
                for b, (rows, gsem) in enumerate(gbufs):
                    i = t * 2 + b
                    nrows, ngsem = gbufs[1 - b]
                    ssem, nssem = ssems[b], ssems[1 - b]
                    pltpu.make_async_copy(
                        hk_hbm.at[srcv.at[i]], rows, gsem).wait()
                    pltpu.async_copy(rows, acc.at[dstv.at[i]], ssem, add=True)

                    @pl.when((i + 1 < HALF) & (i >= 1))
                    def _():
                        pltpu.make_async_copy(
                            rows, acc.at[dstv.at[i]], nssem).wait()

                    @pl.when(i + 1 < HALF)
                    def _():
                        pltpu.async_copy(
                            hk_hbm.at[srcv.at[i + 1]], nrows, ngsem)
                return carry

            lax.fori_loop(0, HALF // 2, body, 0)
            pltpu.make_async_copy(rows0, acc.at[dstv.at[0]], ssem0).wait()
            pltpu.make_async_copy(rows1, acc.at[dstv.at[0]], ssem1).wait()
        plsc.subcore_barrier()
        pltpu.sync_copy(
            acc.at[pl.ds(base_row, ROWS_PER_TILE)],
            out_hbm.at[cid, pl.ds(base_row, ROWS_PER_TILE)],
        )

    return sc_degree, sc_aggregate


BLK = 2000


def _maxk_rows(h):
    work = h
    t = None
    for _ in range(K):
        t = jnp.max(work, axis=1, keepdims=True)
        work = jnp.where(work >= t, -jnp.inf, work)
    return jnp.where(h >= t, h, 0.0)


def _k0_body(x_ref, w_ref, b_ref, o_ref):
    h = jnp.dot(x_ref[...], w_ref[...], preferred_element_type=jnp.float32)
    o_ref[...] = _maxk_rows(h + b_ref[...])


def _agg_scaled(p0_ref, p1_ref, d0_ref, d1_ref):
    deg = d0_ref[...][0, :, 0:1] + d1_ref[...][0, :, 0:1]
    agg = p0_ref[...][0] + p1_ref[...][0]
    return agg / jnp.maximum(deg, 1.0)


def _k1_body(hk_ref, p0_ref, p1_ref, d0_ref, d1_ref,
             ws_ref, wn_ref, b_ref, o_ref):
    agg = _agg_scaled(p0_ref, p1_ref, d0_ref, d1_ref)
    h = (jnp.dot(hk_ref[...], ws_ref[...], preferred_element_type=jnp.float32)
         + jnp.dot(agg, wn_ref[...], preferred_element_type=jnp.float32)
         + b_ref[...])
    o_ref[...] = _maxk_rows(h)


def _k2_body(hk_ref, p0_ref, p1_ref, d0_ref, d1_ref,
             ws_ref, wn_ref, b_ref, wo_ref, bo_ref, o_ref):
    agg = _agg_scaled(p0_ref, p1_ref, d0_ref, d1_ref)
    h = (jnp.dot(hk_ref[...], ws_ref[...], preferred_element_type=jnp.float32)
         + jnp.dot(agg, wn_ref[...], preferred_element_type=jnp.float32)
         + b_ref[...])
    o_ref[...] = (jnp.dot(h, wo_ref[...], preferred_element_type=jnp.float32)
                  + bo_ref[...])


def _row_spec():
    return pl.BlockSpec((BLK, D), lambda i: (i, 0))


def _full_spec(shape):
    return pl.BlockSpec(shape, lambda i: tuple(0 for _ in shape))


def _tc_call(body, num_inputs_rowwise, num_full, full_shapes):
    in_specs = [_row_spec() for _ in range(num_inputs_rowwise)]
    in_specs += [_full_spec(s) for s in full_shapes]
    return pl.pallas_call(
        body,
        grid=(N // BLK,),
        in_specs=in_specs,
        out_specs=_row_spec(),
        out_shape=jax.ShapeDtypeStruct((N, D), jnp.float32),
    )


def kernel(x, edge_index, W_in, b_in, W_self, W_neigh, b_neigh, W_out, b_out):
    idx_pad = N + jnp.arange(EPAD - E, dtype=jnp.int32) % (NP - N)
    src = jnp.concatenate([edge_index[0].astype(jnp.int32), idx_pad])
    src = src.reshape(NCHUNK, CHUNK)
    dst = jnp.concatenate([edge_index[1].astype(jnp.int32), idx_pad])
    dst = dst.reshape(NCHUNK, CHUNK)

    ones_deg = jnp.ones((CHUNK, DEGW), jnp.float32)
    zeros_deg = jnp.zeros((ROWS_PER_TILE, DEGW), jnp.float32)
    zeros_agg = jnp.zeros((ROWS_PER_TILE, D), jnp.float32)

    sc_degree, sc_aggregate = _sc_kernels()
    degp = sc_degree(dst, ones_deg, zeros_deg)

    b_in2 = b_in.reshape(1, D)
    bo2 = b_out.reshape(1, D)

    def _pc_spec(c, w):
        return pl.BlockSpec((1, BLK, w), lambda i, c=c: (c, i, 0))

    k0 = pl.pallas_call(
        _k0_body,
        grid=(N // BLK,),
        in_specs=[pl.BlockSpec((BLK, D), lambda i: (i, 0)),
                  _full_spec((D, D)), _full_spec((1, D))],
        out_specs=_row_spec(),
        out_shape=jax.ShapeDtypeStruct((NP, D), jnp.float32),
    )
    hk = k0(x, W_in, b_in2)

    k1 = pl.pallas_call(
        _k1_body,
        grid=(N // BLK,),
        in_specs=[_row_spec(), _pc_spec(0, D), _pc_spec(1, D),
                  _pc_spec(0, DEGW), _pc_spec(1, DEGW),
                  _full_spec((D, D)), _full_spec((D, D)), _full_spec((1, D))],
        out_specs=_row_spec(),
        out_shape=jax.ShapeDtypeStruct((NP, D), jnp.float32),
    )
    k2 = pl.pallas_call(
        _k2_body,
        grid=(N // BLK,),
        in_specs=[_row_spec(), _pc_spec(0, D), _pc_spec(1, D),
                  _pc_spec(0, DEGW), _pc_spec(1, DEGW),
                  _full_spec((D, D)), _full_spec((D, D)), _full_spec((1, D)),
                  _full_spec((D, D)), _full_spec((1, D))],
        out_specs=pl.BlockSpec((BLK, D), lambda i: (i, 0)),
        out_shape=jax.ShapeDtypeStruct((N, D), jnp.float32),
    )

    for l in range(L):
        p = sc_aggregate(hk, src, dst, zeros_agg)
        bl = b_neigh[l].reshape(1, D)
        if l < L - 1:
            hk = k1(hk, p, p, degp, degp, W_self[l], W_neigh[l], bl)
        else:
            out = k2(hk, p, p, degp, degp, W_self[l], W_neigh[l], bl,
                     W_out, bo2)
    return out
